# baseline (speedup 1.0000x reference)
import jax
import jax.numpy as jnp
from jax import lax
from jax.experimental import pallas as pl
from jax.experimental.pallas import tpu as pltpu
from jax.experimental.pallas import tpu_sc as plsc

S = 100000
B = 4096
D = 128
NEG = 256
TEMP = 0.07
MOM = 0.5
INV_T = 1.0 / TEMP

NC = 2
NS = 16
NW = NC * NS
IPW = B // NW
HALF = 128
UNITS = IPW * (NEG // HALF)
LANES = 16
VPD = D // LANES


def _l2norm(x):
    n = jnp.sqrt(jnp.sum(x * x, axis=1, keepdims=True))
    return x / jnp.maximum(n, 1e-12)


def _norm_body(a_ref, v_ref, na_ref, nv_ref):
    na_ref[...] = _l2norm(a_ref[...])
    nv_ref[...] = _l2norm(v_ref[...])


def _norms(audio, video):
    return pl.pallas_call(
        _norm_body,
        out_shape=(jax.ShapeDtypeStruct((B, D), jnp.float32),
                   jax.ShapeDtypeStruct((B, D), jnp.float32)),
    )(audio, video)


def _head_body(na, nv, pa, pv, o1, o2, o3, o4, ua, uv):
    naf, nvf, paf, pvf = na[...], nv[...], pa[...], pv[...]
    o1[...] = jnp.sum(naf * pvf, axis=1, keepdims=True) * INV_T
    o2[...] = jnp.sum(nvf * paf, axis=1, keepdims=True) * INV_T
    o3[...] = jnp.sum(naf * paf, axis=1, keepdims=True) * INV_T
    o4[...] = jnp.sum(nvf * pvf, axis=1, keepdims=True) * INV_T
    ua[...] = _l2norm(paf * MOM + naf * (1.0 - MOM))
    uv[...] = _l2norm(pvf * MOM + nvf * (1.0 - MOM))


def _head(norm_a, norm_v, pos_a, pos_v):
    return pl.pallas_call(
        _head_body,
        out_shape=(jax.ShapeDtypeStruct((B, 1), jnp.float32),
                   jax.ShapeDtypeStruct((B, 1), jnp.float32),
                   jax.ShapeDtypeStruct((B, 1), jnp.float32),
                   jax.ShapeDtypeStruct((B, 1), jnp.float32),
                   jax.ShapeDtypeStruct((B, D), jnp.float32),
                   jax.ShapeDtypeStruct((B, D), jnp.float32)),
    )(norm_a, norm_v, pos_a, pos_v)


def _sc_mesh():
    return plsc.VectorSubcoreMesh(core_axis_name="c", subcore_axis_name="s",
                                  num_cores=NC, num_subcores=NS)


def _negdot_body(mem16_hbm, memf_hbm, negidx_hbm, ctx1_hbm, ctx2_hbm,
                 posidx_hbm,
                 s1_hbm, s2_hbm, pos_hbm,
                 negidx_v, ctx1_v, ctx2_v, posidx_v, posrows_v,
                 rows_0, rows_1, rows_2, rows_3, s1p0, s1p1, s2p0, s2p1,
                 part1_v, part2_v,
                 gsem_0, gsem_1, gsem_2, gsem_3, osem0, osem1, msem):
    wid = lax.axis_index("s") * NC + lax.axis_index("c")
    base = wid * IPW

    pltpu.sync_copy(negidx_hbm.at[wid], negidx_v)
    pltpu.sync_copy(ctx1_hbm.at[pl.ds(base, IPW)], ctx1_v)
    pltpu.sync_copy(ctx2_hbm.at[pl.ds(base, IPW)], ctx2_v)
    pltpu.sync_copy(posidx_hbm.at[wid], posidx_v)

    rows = (rows_0, rows_1, rows_2, rows_3)
    gsems = (gsem_0, gsem_1, gsem_2, gsem_3)
    s1bufs = (s1p0, s1p1)
    s2bufs = (s2p0, s2p1)
    osems = (osem0, osem1)

    def gather_start(u, buf):
        pltpu.make_async_copy(mem16_hbm.at[negidx_v.at[u]], rows[buf],
                              gsems[buf]).start()

    def gather_wait(buf):
        pltpu.make_async_copy(mem16_hbm.at[negidx_v.at[0]], rows[buf],
                              gsems[buf]).wait()

    for u0 in range(4):
        gather_start(u0, u0)

    pos_cp = pltpu.make_async_copy(memf_hbm.at[posidx_v], posrows_v, msem)
    pos_cp.start()
    pos_cp.wait()
    pltpu.sync_copy(posrows_v, pos_hbm.at[pl.ds(base, IPW)])

    @pl.loop(0, IPW // 2)
    def _pair(k):
        for sub in range(2):
            item = 2 * k + sub
            par = sub
            s1b, s2b, osem = s1bufs[par], s2bufs[par], osems[par]

            @pl.when(item >= 2)
            def _drain():
                pltpu.make_async_copy(s1b, s1_hbm.at[0], osem).wait()
                pltpu.make_async_copy(s2b, s2_hbm.at[0], osem).wait()

            c1 = [ctx1_v[item, pl.ds(LANES * j, LANES)] for j in range(VPD)]
            c2 = [ctx2_v[item, pl.ds(LANES * j, LANES)] for j in range(VPD)]

            for h in range(2):
                u = 2 * item + h
                gbuf = 2 * sub + h
                gather_wait(gbuf)
                rbuf = rows[gbuf]

                @pl.loop(0, HALF, unroll=4)
                def _row(r):
                    t = []
                    for q in range(VPD // 2):
                        pk32 = rbuf[r, pl.ds(LANES * q, LANES)]
                        pk = plsc.bitcast(pk32, jnp.bfloat16)
                        ev, od = plsc.unpack(
                            pk, format=plsc.PackFormat.INTERLEAVED)
                        t.append((ev, od))
                    for cs, dst in ((c1, part1_v), (c2, part2_v)):
                        p = [t[q][half] * cs[2 * q + half]
                             for q in range(VPD // 2) for half in range(2)]
                        u0, u1 = p[0] + p[1], p[2] + p[3]
                        u2, u3 = p[4] + p[5], p[6] + p[7]
                        dst[pl.ds(r * LANES, LANES)] = (u0 + u1) + (u2 + u3)

                lane = lax.broadcasted_iota(jnp.int32, (LANES,), 0)
                for g in range(HALF // LANES):
                    fbase = lane * LANES + (g * LANES * LANES)
                    for src, sb in ((part1_v, s1b), (part2_v, s2b)):
                        acc = [plsc.load_gather(src, [fbase + c])
                               for c in range(4)]
                        for c in range(4, LANES):
                            acc[c % 4] = acc[c % 4] + plsc.load_gather(
                                src, [fbase + c])
                        sb[h, pl.ds(g * LANES, LANES)] = (
                            (acc[0] + acc[1]) + (acc[2] + acc[3])) * INV_T

                nxt = u + 4

                @pl.when(nxt < UNITS)
                def _refill():
                    gather_start(nxt, gbuf)

            gitem = base + item
            pltpu.make_async_copy(s1b, s1_hbm.at[gitem], osem).start()
            pltpu.make_async_copy(s2b, s2_hbm.at[gitem], osem).start()

    for par in range(2):
        pltpu.make_async_copy(s1bufs[par], s1_hbm.at[0], osems[par]).wait()
        pltpu.make_async_copy(s2bufs[par], s2_hbm.at[0], osems[par]).wait()


def _negdot(mem16, memf, negidx3, ctx1p, ctx2p, posidx2):
    fn = pl.kernel(
        _negdot_body,
        out_type=(jax.ShapeDtypeStruct((B, 2, HALF), jnp.float32),
                  jax.ShapeDtypeStruct((B, 2, HALF), jnp.float32),
                  jax.ShapeDtypeStruct((B, D), jnp.float32)),
        mesh=_sc_mesh(),
        compiler_params=pltpu.CompilerParams(needs_layout_passes=False,
                                             use_tc_tiling_on_sc=False),
        scratch_types=[
            pltpu.VMEM((UNITS, HALF), jnp.int32),
            pltpu.VMEM((IPW, D), jnp.float32),
            pltpu.VMEM((IPW, D), jnp.float32),
            pltpu.VMEM((IPW,), jnp.int32),
            pltpu.VMEM((IPW, D), jnp.float32),
            pltpu.VMEM((HALF, D // 2), jnp.int32),
            pltpu.VMEM((HALF, D // 2), jnp.int32),
            pltpu.VMEM((HALF, D // 2), jnp.int32),
            pltpu.VMEM((HALF, D // 2), jnp.int32),
            pltpu.VMEM((2, HALF), jnp.float32),
            pltpu.VMEM((2, HALF), jnp.float32),
            pltpu.VMEM((2, HALF), jnp.float32),
            pltpu.VMEM((2, HALF), jnp.float32),
            pltpu.VMEM((HALF * LANES,), jnp.float32),
            pltpu.VMEM((HALF * LANES,), jnp.float32),
            pltpu.SemaphoreType.DMA,
            pltpu.SemaphoreType.DMA,
            pltpu.SemaphoreType.DMA,
            pltpu.SemaphoreType.DMA,
            pltpu.SemaphoreType.DMA,
            pltpu.SemaphoreType.DMA,
            pltpu.SemaphoreType.DMA,
        ],
    )
    return fn(mem16, memf, negidx3, ctx1p, ctx2p, posidx2)


def _scatter_body(upd_a_hbm, upd_v_hbm, w_hbm, t_hbm, mema_ref, memv_ref,
                  w_v, t_v, rows_v, sem):
    wid = lax.axis_index("s") * NC + lax.axis_index("c")
    pltpu.sync_copy(w_hbm.at[wid], w_v)
    pltpu.sync_copy(t_hbm.at[wid], t_v)
    for upd, memref in ((upd_a_hbm, mema_ref), (upd_v_hbm, memv_ref)):
        cp = pltpu.make_async_copy(upd.at[w_v], rows_v, sem)
        cp.start()
        cp.wait()
        cp2 = pltpu.make_async_copy(rows_v, memref.at[t_v], sem)
        cp2.start()
        cp2.wait()


def _scatter(upd_a, upd_v, w2, t2, mema_ref, memv_ref):
    fn = pl.kernel(
        _scatter_body,
        out_type=(),
        mesh=_sc_mesh(),
        scratch_types=[
            pltpu.VMEM((IPW,), jnp.int32),
            pltpu.VMEM((IPW,), jnp.int32),
            pltpu.VMEM((IPW, D), jnp.float32),
            pltpu.SemaphoreType.DMA,
        ],
    )
    return fn(upd_a, upd_v, w2, t2, mema_ref, memv_ref)


def kernel(audio, video, indices, audio_memory, video_memory):
    key = jax.random.key(42)
    ka, kv = jax.random.split(key)

    def _neg(k):
        s = jax.random.randint(k, (B, NEG), 0, S - 1, dtype=jnp.int32)
        return s + (s >= indices[:, None]).astype(jnp.int32)

    negidx_a = _neg(ka)
    negidx_v = _neg(kv)

    norm_a, norm_v = _norms(audio, video)

    posidx2 = indices.reshape(NW, IPW)
    n3a = negidx_a.reshape(NW, UNITS, HALF)
    n3v = negidx_v.reshape(NW, UNITS, HALF)

    def _to16(m):
        m16 = m.astype(jnp.bfloat16).reshape(S, D // 2, 2)
        return lax.bitcast_convert_type(m16, jnp.int32)

    mem16_a = _to16(audio_memory)
    mem16_v = _to16(video_memory)

    def _perm(c):
        return c.reshape(B, 4, LANES, 2).transpose(0, 1, 3, 2).reshape(B, D)

    norm_a_p = _perm(norm_a)
    norm_v_p = _perm(norm_v)

    s1v, s2v, pos_v = _negdot(mem16_v, video_memory, n3v,
                              norm_a_p, norm_v_p, posidx2)
    s1a, s2a, pos_a = _negdot(mem16_a, audio_memory, n3a,
                              norm_v_p, norm_a_p, posidx2)

    o1, o2, o3, o4, upd_a, upd_v = _head(norm_a, norm_v, pos_a, pos_v)

    iota = jnp.arange(B, dtype=jnp.int32)
    lastw = jnp.zeros((S,), jnp.int32).at[indices].max(iota)
    w = lastw[indices]
    w2 = w.reshape(NW, IPW)
    t2 = indices.reshape(NW, IPW)

    mema_ref = jax.new_ref(audio_memory)
    memv_ref = jax.new_ref(video_memory)
    _scatter(upd_a, upd_v, w2, t2, mema_ref, memv_ref)
    out_a = mema_ref[...]
    out_v = memv_ref[...]

    return (o1, s1v.reshape(B, NEG), o2, s1a.reshape(B, NEG),
            o3, s2a.reshape(B, NEG), o4, s2v.reshape(B, NEG),
            out_a, out_v)


# speedup vs baseline: 1.6414x; 1.6414x over previous
import jax
import jax.numpy as jnp
from jax import lax
from jax.experimental import pallas as pl
from jax.experimental.pallas import tpu as pltpu
from jax.experimental.pallas import tpu_sc as plsc

S = 100000
B = 4096
D = 128
NEG = 256
TEMP = 0.07
MOM = 0.5
INV_T = 1.0 / TEMP

NC = 2
NS = 16
NW = NC * NS
IPW = B // NW
HALF = 128
UNITS = IPW * (NEG // HALF)
LANES = 16
VPD = D // LANES


def _l2norm(x):
    n = jnp.sqrt(jnp.sum(x * x, axis=1, keepdims=True))
    return x / jnp.maximum(n, 1e-12)


def _norm_body(a_ref, v_ref, na_ref, nv_ref):
    na_ref[...] = _l2norm(a_ref[...])
    nv_ref[...] = _l2norm(v_ref[...])


def _norms(audio, video):
    return pl.pallas_call(
        _norm_body,
        out_shape=(jax.ShapeDtypeStruct((B, D), jnp.float32),
                   jax.ShapeDtypeStruct((B, D), jnp.float32)),
    )(audio, video)


def _head_body(na, nv, pa, pv, o1, o2, o3, o4, ua, uv):
    naf, nvf, paf, pvf = na[...], nv[...], pa[...], pv[...]
    o1[...] = jnp.sum(naf * pvf, axis=1, keepdims=True) * INV_T
    o2[...] = jnp.sum(nvf * paf, axis=1, keepdims=True) * INV_T
    o3[...] = jnp.sum(naf * paf, axis=1, keepdims=True) * INV_T
    o4[...] = jnp.sum(nvf * pvf, axis=1, keepdims=True) * INV_T
    ua[...] = _l2norm(paf * MOM + naf * (1.0 - MOM))
    uv[...] = _l2norm(pvf * MOM + nvf * (1.0 - MOM))


def _head(norm_a, norm_v, pos_a, pos_v):
    return pl.pallas_call(
        _head_body,
        out_shape=(jax.ShapeDtypeStruct((B, 1), jnp.float32),
                   jax.ShapeDtypeStruct((B, 1), jnp.float32),
                   jax.ShapeDtypeStruct((B, 1), jnp.float32),
                   jax.ShapeDtypeStruct((B, 1), jnp.float32),
                   jax.ShapeDtypeStruct((B, D), jnp.float32),
                   jax.ShapeDtypeStruct((B, D), jnp.float32)),
    )(norm_a, norm_v, pos_a, pos_v)


def _sc_mesh():
    return plsc.VectorSubcoreMesh(core_axis_name="c", subcore_axis_name="s",
                                  num_cores=NC, num_subcores=NS)


def _make_negdot_body(with_pos):
    def body(*refs):
        if with_pos:
            (mem_hbm, memo_hbm, negidx_hbm, ctx1_hbm, ctx2_hbm, posidx_hbm,
             s1_hbm, s2_hbm, posm_hbm, poso_hbm,
             negidx_v, ctx1_v, ctx2_v, posidx_v,
             rows_a, rows_b, s1p0, s1p1, s2p0, s2p1,
             part1_v, part2_v,
             gsem_a, gsem_b, osem0, osem1, msem) = refs
        else:
            (mem_hbm, negidx_hbm, ctx1_hbm, ctx2_hbm,
             s1_hbm, s2_hbm,
             negidx_v, ctx1_v, ctx2_v, posidx_v,
             rows_a, rows_b, s1p0, s1p1, s2p0, s2p1,
             part1_v, part2_v,
             gsem_a, gsem_b, osem0, osem1, msem) = refs
        wid = lax.axis_index("s") * NC + lax.axis_index("c")
        base = wid * IPW

        pltpu.sync_copy(negidx_hbm.at[wid], negidx_v)
        pltpu.sync_copy(ctx1_hbm.at[pl.ds(base, IPW)], ctx1_v)
        pltpu.sync_copy(ctx2_hbm.at[pl.ds(base, IPW)], ctx2_v)

        if with_pos:
            pltpu.sync_copy(posidx_hbm.at[wid], posidx_v)
            for src, dst in ((mem_hbm, posm_hbm), (memo_hbm, poso_hbm)):
                cp = pltpu.make_async_copy(src.at[posidx_v], rows_a, msem)
                cp.start()
                cp.wait()
                pltpu.sync_copy(rows_a, dst.at[pl.ds(base, IPW)])

        rows = (rows_a, rows_b)
        gsems = (gsem_a, gsem_b)
        s1bufs = (s1p0, s1p1)
        s2bufs = (s2p0, s2p1)
        osems = (osem0, osem1)

        def gather_start(u, buf):
            pltpu.make_async_copy(mem_hbm.at[negidx_v.at[u]], rows[buf],
                                  gsems[buf]).start()

        def gather_wait(buf):
            pltpu.make_async_copy(mem_hbm.at[negidx_v.at[0]], rows[buf],
                                  gsems[buf]).wait()

        gather_start(0, 0)
        gather_start(1, 1)

        @pl.loop(0, IPW // 2)
        def _pair(k):
            for sub in range(2):
                item = 2 * k + sub
                par = sub
                s1b, s2b, osem = s1bufs[par], s2bufs[par], osems[par]

                @pl.when(item >= 2)
                def _drain():
                    pltpu.make_async_copy(s1b, s1_hbm.at[0], osem).wait()
                    pltpu.make_async_copy(s2b, s2_hbm.at[0], osem).wait()

                c1 = [ctx1_v[item, pl.ds(LANES * j, LANES)]
                      for j in range(VPD)]
                c2 = [ctx2_v[item, pl.ds(LANES * j, LANES)]
                      for j in range(VPD)]

                for h in range(2):
                    u = 2 * item + h
                    gbuf = h
                    gather_wait(gbuf)
                    rbuf = rows[gbuf]

                    @pl.loop(0, HALF, unroll=4)
                    def _row(r):
                        rv = [rbuf[r, pl.ds(LANES * j, LANES)]
                              for j in range(VPD)]
                        for cs, dst in ((c1, part1_v), (c2, part2_v)):
                            t = [rv[j] * cs[j] for j in range(VPD)]
                            u0, u1 = t[0] + t[1], t[2] + t[3]
                            u2, u3 = t[4] + t[5], t[6] + t[7]
                            dst[pl.ds(r * LANES, LANES)] = (
                                (u0 + u1) + (u2 + u3))

                    lane = lax.broadcasted_iota(jnp.int32, (LANES,), 0)
                    for g in range(HALF // LANES):
                        fbase = lane * LANES + (g * LANES * LANES)
                        for src, sb in ((part1_v, s1b), (part2_v, s2b)):
                            acc = [plsc.load_gather(src, [fbase + c])
                                   for c in range(4)]
                            for c in range(4, LANES):
                                acc[c % 4] = acc[c % 4] + plsc.load_gather(
                                    src, [fbase + c])
                            sb[h, pl.ds(g * LANES, LANES)] = (
                                (acc[0] + acc[1]) + (acc[2] + acc[3])) * INV_T

                    nxt = u + 2

                    @pl.when(nxt < UNITS)
                    def _refill():
                        gather_start(nxt, gbuf)

                gitem = base + item
                pltpu.make_async_copy(s1b, s1_hbm.at[gitem], osem).start()
                pltpu.make_async_copy(s2b, s2_hbm.at[gitem], osem).start()

        for par in range(2):
            pltpu.make_async_copy(s1bufs[par], s1_hbm.at[0],
                                  osems[par]).wait()
            pltpu.make_async_copy(s2bufs[par], s2_hbm.at[0],
                                  osems[par]).wait()

    return body


_SC_SCRATCH = [
    pltpu.VMEM((UNITS, HALF), jnp.int32),
    pltpu.VMEM((IPW, D), jnp.float32),
    pltpu.VMEM((IPW, D), jnp.float32),
    pltpu.VMEM((IPW,), jnp.int32),
    pltpu.VMEM((HALF, D), jnp.float32),
    pltpu.VMEM((HALF, D), jnp.float32),
    pltpu.VMEM((2, HALF), jnp.float32),
    pltpu.VMEM((2, HALF), jnp.float32),
    pltpu.VMEM((2, HALF), jnp.float32),
    pltpu.VMEM((2, HALF), jnp.float32),
    pltpu.VMEM((HALF * LANES,), jnp.float32),
    pltpu.VMEM((HALF * LANES,), jnp.float32),
    pltpu.SemaphoreType.DMA,
    pltpu.SemaphoreType.DMA,
    pltpu.SemaphoreType.DMA,
    pltpu.SemaphoreType.DMA,
    pltpu.SemaphoreType.DMA,
]

_SCORES = jax.ShapeDtypeStruct((B, 2, HALF), jnp.float32)
_POS = jax.ShapeDtypeStruct((B, D), jnp.float32)


def _negdot_pos(mem, memo, negidx3, ctx1, ctx2, posidx2):
    fn = pl.kernel(
        _make_negdot_body(True),
        out_type=(_SCORES, _SCORES, _POS, _POS),
        mesh=_sc_mesh(),
        compiler_params=pltpu.CompilerParams(needs_layout_passes=False),
        scratch_types=_SC_SCRATCH,
    )
    return fn(mem, memo, negidx3, ctx1, ctx2, posidx2)


def _negdot(mem, negidx3, ctx1, ctx2):
    fn = pl.kernel(
        _make_negdot_body(False),
        out_type=(_SCORES, _SCORES),
        mesh=_sc_mesh(),
        compiler_params=pltpu.CompilerParams(needs_layout_passes=False),
        scratch_types=_SC_SCRATCH,
    )
    return fn(mem, negidx3, ctx1, ctx2)


def _scatter_body(upd_a_hbm, upd_v_hbm, w_hbm, t_hbm, mema_ref, memv_ref,
                  w_v, t_v, rows_v, sem):
    wid = lax.axis_index("s") * NC + lax.axis_index("c")
    pltpu.sync_copy(w_hbm.at[wid], w_v)
    pltpu.sync_copy(t_hbm.at[wid], t_v)
    for upd, memref in ((upd_a_hbm, mema_ref), (upd_v_hbm, memv_ref)):
        cp = pltpu.make_async_copy(upd.at[w_v], rows_v, sem)
        cp.start()
        cp.wait()
        cp2 = pltpu.make_async_copy(rows_v, memref.at[t_v], sem)
        cp2.start()
        cp2.wait()


def _scatter(upd_a, upd_v, w2, t2, mema_ref, memv_ref):
    fn = pl.kernel(
        _scatter_body,
        out_type=(),
        mesh=_sc_mesh(),
        scratch_types=[
            pltpu.VMEM((IPW,), jnp.int32),
            pltpu.VMEM((IPW,), jnp.int32),
            pltpu.VMEM((IPW, D), jnp.float32),
            pltpu.SemaphoreType.DMA,
        ],
    )
    return fn(upd_a, upd_v, w2, t2, mema_ref, memv_ref)


def kernel(audio, video, indices, audio_memory, video_memory):
    key = jax.random.key(42)
    ka, kv = jax.random.split(key)

    def _neg(k):
        s = jax.random.randint(k, (B, NEG), 0, S - 1, dtype=jnp.int32)
        return s + (s >= indices[:, None]).astype(jnp.int32)

    negidx_a = _neg(ka)
    negidx_v = _neg(kv)

    norm_a, norm_v = _norms(audio, video)

    posidx2 = indices.reshape(NW, IPW)
    n3a = negidx_a.reshape(NW, UNITS, HALF)
    n3v = negidx_v.reshape(NW, UNITS, HALF)

    s1v, s2v, pos_v, pos_a = _negdot_pos(video_memory, audio_memory, n3v,
                                         norm_a, norm_v, posidx2)
    o1, o2, o3, o4, upd_a, upd_v = _head(norm_a, norm_v, pos_a, pos_v)

    iota = jnp.arange(B, dtype=jnp.int32)
    lastw = jnp.zeros((S,), jnp.int32).at[indices].max(iota)
    w = lastw[indices]
    w2 = w.reshape(NW, IPW)
    t2 = indices.reshape(NW, IPW)

    s1a, s2a = _negdot(audio_memory, n3a, norm_v, norm_a)

    mema_ref = jax.new_ref(audio_memory)
    memv_ref = jax.new_ref(video_memory)
    _scatter(upd_a, upd_v, w2, t2, mema_ref, memv_ref)
    out_a = mema_ref[...]
    out_v = memv_ref[...]

    return (o1, s1v.reshape(B, NEG), o2, s1a.reshape(B, NEG),
            o3, s2a.reshape(B, NEG), o4, s2v.reshape(B, NEG),
            out_a, out_v)


# speedup vs baseline: 1.7061x; 1.0395x over previous
import jax
import jax.numpy as jnp
from jax import lax
from jax.experimental import pallas as pl
from jax.experimental.pallas import tpu as pltpu
from jax.experimental.pallas import tpu_sc as plsc

S = 100000
B = 4096
D = 128
NEG = 256
TEMP = 0.07
MOM = 0.5
INV_T = 1.0 / TEMP

NC = 2
NS = 16
NW = NC * NS
IPW = B // NW
HALF = 128
UNITS = IPW * (NEG // HALF)
LANES = 16
VPD = D // LANES


def _l2norm(x):
    n = jnp.sqrt(jnp.sum(x * x, axis=1, keepdims=True))
    return x / jnp.maximum(n, 1e-12)


def _norm_body(a_ref, v_ref, na_ref, nv_ref):
    na_ref[...] = _l2norm(a_ref[...])
    nv_ref[...] = _l2norm(v_ref[...])


def _norms(audio, video):
    return pl.pallas_call(
        _norm_body,
        out_shape=(jax.ShapeDtypeStruct((B, D), jnp.float32),
                   jax.ShapeDtypeStruct((B, D), jnp.float32)),
    )(audio, video)


def _head_body(na, nv, pa, pv, o1, o2, o3, o4, ua, uv):
    naf, nvf, paf, pvf = na[...], nv[...], pa[...], pv[...]
    o1[...] = jnp.sum(naf * pvf, axis=1, keepdims=True) * INV_T
    o2[...] = jnp.sum(nvf * paf, axis=1, keepdims=True) * INV_T
    o3[...] = jnp.sum(naf * paf, axis=1, keepdims=True) * INV_T
    o4[...] = jnp.sum(nvf * pvf, axis=1, keepdims=True) * INV_T
    ua[...] = _l2norm(paf * MOM + naf * (1.0 - MOM))
    uv[...] = _l2norm(pvf * MOM + nvf * (1.0 - MOM))


def _head(norm_a, norm_v, pos_a, pos_v):
    return pl.pallas_call(
        _head_body,
        out_shape=(jax.ShapeDtypeStruct((B, 1), jnp.float32),
                   jax.ShapeDtypeStruct((B, 1), jnp.float32),
                   jax.ShapeDtypeStruct((B, 1), jnp.float32),
                   jax.ShapeDtypeStruct((B, 1), jnp.float32),
                   jax.ShapeDtypeStruct((B, D), jnp.float32),
                   jax.ShapeDtypeStruct((B, D), jnp.float32)),
    )(norm_a, norm_v, pos_a, pos_v)


def _wlast_body(idx2, w2):
    idx_all = idx2[...].reshape(1, B)
    iota_all = lax.broadcasted_iota(jnp.int32, (1, B), 1)

    @pl.loop(0, NW)
    def _blk(i):
        idx_blk = idx2[i, :].reshape(IPW, 1)
        cand = jnp.where(idx_blk == idx_all, iota_all, -1)
        w2[i, :] = jnp.max(cand, axis=1)


def _wlast(idx2):
    return pl.pallas_call(
        _wlast_body,
        out_shape=jax.ShapeDtypeStruct((NW, IPW), jnp.int32),
    )(idx2)


def _sc_mesh():
    return plsc.VectorSubcoreMesh(core_axis_name="c", subcore_axis_name="s",
                                  num_cores=NC, num_subcores=NS)


def _negdot_all_body(memv_hbm, mema_hbm, negv_hbm, nega_hbm,
                     ctxna_hbm, ctxnv_hbm, posidx_hbm,
                     s1v_hbm, s2v_hbm, s1a_hbm, s2a_hbm, posv_hbm, posa_hbm,
                     negidx_v, ctxna_v, ctxnv_v, posidx_v,
                     rows_a, rows_b, s1p0, s1p1, s2p0, s2p1,
                     part1_v, part2_v,
                     gsem_a, gsem_b, osem0, osem1, msem):
    wid = lax.axis_index("s") * NC + lax.axis_index("c")
    base = wid * IPW

    pltpu.sync_copy(ctxna_hbm.at[pl.ds(base, IPW)], ctxna_v)
    pltpu.sync_copy(ctxnv_hbm.at[pl.ds(base, IPW)], ctxnv_v)
    pltpu.sync_copy(posidx_hbm.at[wid], posidx_v)

    for src, dst in ((memv_hbm, posv_hbm), (mema_hbm, posa_hbm)):
        cp = pltpu.make_async_copy(src.at[posidx_v], rows_a, msem)
        cp.start()
        cp.wait()
        pltpu.sync_copy(rows_a, dst.at[pl.ds(base, IPW)])

    rows = (rows_a, rows_b)
    gsems = (gsem_a, gsem_b)
    s1bufs = (s1p0, s1p1)
    s2bufs = (s2p0, s2p1)
    osems = (osem0, osem1)

    for mem_hbm, negidx_hbm, ctx1_v, ctx2_v, s1_hbm, s2_hbm in (
            (memv_hbm, negv_hbm, ctxna_v, ctxnv_v, s1v_hbm, s2v_hbm),
            (mema_hbm, nega_hbm, ctxnv_v, ctxna_v, s1a_hbm, s2a_hbm)):
        pltpu.sync_copy(negidx_hbm.at[wid], negidx_v)

        def gather_start(u, buf, _mem=mem_hbm):
            pltpu.make_async_copy(_mem.at[negidx_v.at[u]], rows[buf],
                                  gsems[buf]).start()

        def gather_wait(buf, _mem=mem_hbm):
            pltpu.make_async_copy(_mem.at[negidx_v.at[0]], rows[buf],
                                  gsems[buf]).wait()

        gather_start(0, 0)
        gather_start(1, 1)

        @pl.loop(0, IPW // 2)
        def _pair(k, _gs=gather_start, _gw=gather_wait, _c1v=ctx1_v,
                  _c2v=ctx2_v, _s1h=s1_hbm, _s2h=s2_hbm):
            for sub in range(2):
                item = 2 * k + sub
                par = sub
                s1b, s2b, osem = s1bufs[par], s2bufs[par], osems[par]

                @pl.when(item >= 2)
                def _drain():
                    pltpu.make_async_copy(s1b, _s1h.at[0], osem).wait()
                    pltpu.make_async_copy(s2b, _s2h.at[0], osem).wait()

                c1 = [_c1v[item, pl.ds(LANES * j, LANES)]
                      for j in range(VPD)]
                c2 = [_c2v[item, pl.ds(LANES * j, LANES)]
                      for j in range(VPD)]

                for h in range(2):
                    u = 2 * item + h
                    gbuf = h
                    _gw(gbuf)
                    rbuf = rows[gbuf]

                    @pl.loop(0, HALF, unroll=4)
                    def _row(r):
                        rv = [rbuf[r, pl.ds(LANES * j, LANES)]
                              for j in range(VPD)]
                        for cs, dst in ((c1, part1_v), (c2, part2_v)):
                            t = [rv[j] * cs[j] for j in range(VPD)]
                            u0, u1 = t[0] + t[1], t[2] + t[3]
                            u2, u3 = t[4] + t[5], t[6] + t[7]
                            dst[pl.ds(r * LANES, LANES)] = (
                                (u0 + u1) + (u2 + u3))

                    lane = lax.broadcasted_iota(jnp.int32, (LANES,), 0)
                    for g in range(HALF // LANES):
                        fbase = lane * LANES + (g * LANES * LANES)
                        for src, sb in ((part1_v, s1b), (part2_v, s2b)):
                            acc = [plsc.load_gather(src, [fbase + c])
                                   for c in range(4)]
                            for c in range(4, LANES):
                                acc[c % 4] = acc[c % 4] + plsc.load_gather(
                                    src, [fbase + c])
                            sb[h, pl.ds(g * LANES, LANES)] = (
                                (acc[0] + acc[1]) + (acc[2] + acc[3])) * INV_T

                    nxt = u + 2

                    @pl.when(nxt < UNITS)
                    def _refill():
                        _gs(nxt, gbuf)

                gitem = base + item
                pltpu.make_async_copy(s1b, _s1h.at[gitem], osem).start()
                pltpu.make_async_copy(s2b, _s2h.at[gitem], osem).start()

        for par in range(2):
            pltpu.make_async_copy(s1bufs[par], s1_hbm.at[0],
                                  osems[par]).wait()
            pltpu.make_async_copy(s2bufs[par], s2_hbm.at[0],
                                  osems[par]).wait()


_SC_SCRATCH = [
    pltpu.VMEM((UNITS, HALF), jnp.int32),
    pltpu.VMEM((IPW, D), jnp.float32),
    pltpu.VMEM((IPW, D), jnp.float32),
    pltpu.VMEM((IPW,), jnp.int32),
    pltpu.VMEM((HALF, D), jnp.float32),
    pltpu.VMEM((HALF, D), jnp.float32),
    pltpu.VMEM((2, HALF), jnp.float32),
    pltpu.VMEM((2, HALF), jnp.float32),
    pltpu.VMEM((2, HALF), jnp.float32),
    pltpu.VMEM((2, HALF), jnp.float32),
    pltpu.VMEM((HALF * LANES,), jnp.float32),
    pltpu.VMEM((HALF * LANES,), jnp.float32),
    pltpu.SemaphoreType.DMA,
    pltpu.SemaphoreType.DMA,
    pltpu.SemaphoreType.DMA,
    pltpu.SemaphoreType.DMA,
    pltpu.SemaphoreType.DMA,
]

_SCORES = jax.ShapeDtypeStruct((B, 2, HALF), jnp.float32)
_POS = jax.ShapeDtypeStruct((B, D), jnp.float32)


def _negdot_all(mem_v, mem_a, n3v, n3a, norm_a, norm_v, posidx2):
    fn = pl.kernel(
        _negdot_all_body,
        out_type=(_SCORES, _SCORES, _SCORES, _SCORES, _POS, _POS),
        mesh=_sc_mesh(),
        compiler_params=pltpu.CompilerParams(needs_layout_passes=False),
        scratch_types=_SC_SCRATCH,
    )
    return fn(mem_v, mem_a, n3v, n3a, norm_a, norm_v, posidx2)


def _scatter_body(upd_a_hbm, upd_v_hbm, w_hbm, t_hbm, mema_ref, memv_ref,
                  w_v, t_v, rows_v, sem):
    wid = lax.axis_index("s") * NC + lax.axis_index("c")
    pltpu.sync_copy(w_hbm.at[wid], w_v)
    pltpu.sync_copy(t_hbm.at[wid], t_v)
    for upd, memref in ((upd_a_hbm, mema_ref), (upd_v_hbm, memv_ref)):
        cp = pltpu.make_async_copy(upd.at[w_v], rows_v, sem)
        cp.start()
        cp.wait()
        cp2 = pltpu.make_async_copy(rows_v, memref.at[t_v], sem)
        cp2.start()
        cp2.wait()


def _scatter(upd_a, upd_v, w2, t2, mema_ref, memv_ref):
    fn = pl.kernel(
        _scatter_body,
        out_type=(),
        mesh=_sc_mesh(),
        scratch_types=[
            pltpu.VMEM((IPW,), jnp.int32),
            pltpu.VMEM((IPW,), jnp.int32),
            pltpu.VMEM((IPW, D), jnp.float32),
            pltpu.SemaphoreType.DMA,
        ],
    )
    return fn(upd_a, upd_v, w2, t2, mema_ref, memv_ref)


def kernel(audio, video, indices, audio_memory, video_memory):
    key = jax.random.key(42)
    ka, kv = jax.random.split(key)

    def _neg(k):
        s = jax.random.randint(k, (B, NEG), 0, S - 1, dtype=jnp.int32)
        return s + (s >= indices[:, None]).astype(jnp.int32)

    negidx_a = _neg(ka)
    negidx_v = _neg(kv)

    norm_a, norm_v = _norms(audio, video)

    posidx2 = indices.reshape(NW, IPW)
    n3a = negidx_a.reshape(NW, UNITS, HALF)
    n3v = negidx_v.reshape(NW, UNITS, HALF)

    w2 = _wlast(posidx2)
    t2 = posidx2

    s1v, s2v, s1a, s2a, pos_v, pos_a = _negdot_all(
        video_memory, audio_memory, n3v, n3a, norm_a, norm_v, posidx2)

    o1, o2, o3, o4, upd_a, upd_v = _head(norm_a, norm_v, pos_a, pos_v)

    mema_ref = jax.new_ref(audio_memory)
    memv_ref = jax.new_ref(video_memory)
    _scatter(upd_a, upd_v, w2, t2, mema_ref, memv_ref)
    out_a = mema_ref[...]
    out_v = memv_ref[...]

    return (o1, s1v.reshape(B, NEG), o2, s1a.reshape(B, NEG),
            o3, s2a.reshape(B, NEG), o4, s2v.reshape(B, NEG),
            out_a, out_v)
